# baseline (speedup 1.0000x reference)
import jax
import jax.numpy as jnp
from jax import lax
from jax.experimental import pallas as pl
from jax.experimental.pallas import tpu as pltpu
from jax.experimental.pallas import tpu_sc as plsc

_N = 10000
_VOCAB = 100
_EMBED = 10
_H = 128
_E = 320000

_NW = 32
_NPAD = 10240
_NPW = _NPAD // _NW
_GCH = 80
_EPW = _E // _NW
_TPW = _NPAD // 16


def _mesh():
    return plsc.VectorSubcoreMesh(core_axis_name="c", subcore_axis_name="s")


def _ring(nchk, nbuf, gather_start, gather_wait, scatter):
    for b in range(nbuf):
        gather_start(b, b)
    fg = (nchk - nbuf) // nbuf

    def step(grp, carry):
        for b in range(nbuf):
            t = nbuf * grp + b
            gather_wait(t, b)
            scatter(t, b)
            gather_start(t + nbuf, b)
        return carry

    lax.fori_loop(0, fg, step, 0)
    for t in range(fg * nbuf, nchk):
        b = t % nbuf
        gather_wait(t, b)
        scatter(t, b)
        if t + nbuf < nchk:
            gather_start(t + nbuf, b)


_DCH = 80
_DNCHK = 2 * _EPW // _DCH
_NGRP = _NPAD // 16 // 16


def _sc_deg_emb(xp, ei, table):
    epw = 2 * _EPW

    def body(x_hbm, ei_hbm, table_hbm, dinv_out, emb_out,
             partials, degloc, didx_all, pbuf, dinvbuf, xidx, grows, sem):
        c = lax.axis_index("c")
        s = lax.axis_index("s")
        wid = s * 2 + c
        ones16 = jnp.ones((16,), jnp.float32)

        pltpu.sync_copy(ei_hbm.at[1, pl.ds(s * epw, epw)], didx_all)

        def zero_deg(i, carry):
            degloc[pl.ds(i * 16, 16)] = jnp.zeros((16,), jnp.float32)
            return carry

        lax.fori_loop(0, _NPAD // 16, zero_deg, 0)

        def deg_step(t, carry):
            for j in range(4):
                idxv = didx_all[pl.ds(t * 64 + j * 16, 16)]
                cnt, last = plsc.scan_count(idxv)
                plsc.addupdate_scatter(
                    degloc, [idxv], cnt.astype(jnp.float32), mask=last)
            return carry

        lax.fori_loop(0, epw // 64, deg_step, 0)
        pltpu.sync_copy(degloc, partials.at[s])

        for j in range(_NPW // _GCH):
            b = wid * _NPW + j * _GCH
            pltpu.sync_copy(x_hbm.at[pl.ds(b, _GCH)], xidx)
            pltpu.async_copy(table_hbm.at[xidx], grows, sem).wait()
            pltpu.sync_copy(grows, emb_out.at[pl.ds(b, _GCH)])

        plsc.subcore_barrier()

        for p in range(16):
            pltpu.sync_copy(partials.at[p, pl.ds(s * _TPW, _TPW)],
                            pbuf.at[pl.ds(p * _TPW, _TPW)])

        def dinv_step(grp, carry):
            d = pbuf[pl.ds(grp * 16, 16)]
            for p in range(1, 16):
                d = d + pbuf[pl.ds(p * _TPW + grp * 16, 16)]
            d = d + 1.0
            yi = 1597463007 - jnp.right_shift(plsc.bitcast(d, jnp.int32), 1)
            y = plsc.bitcast(yi, jnp.float32)
            h = d * 0.5
            for _ in range(3):
                y = y * (1.5 - h * y * y)
            dinvbuf[pl.ds(grp * 16, 16)] = y
            return carry

        lax.fori_loop(0, _TPW // 16, dinv_step, 0)

        @pl.when(c == 0)
        def _():
            pltpu.sync_copy(dinvbuf, dinv_out.at[pl.ds(s * _TPW, _TPW)])

    f = pl.kernel(
        body,
        out_type=[jax.ShapeDtypeStruct((_NPAD,), jnp.float32),
                  jax.ShapeDtypeStruct((_NPAD, 16), jnp.float32)],
        mesh=_mesh(),
        compiler_params=pltpu.CompilerParams(
            use_tc_tiling_on_sc=False, needs_layout_passes=False),
        scratch_types=[
            pltpu.VMEM_SHARED((16, _NPAD), jnp.float32),
            pltpu.VMEM((_NPAD,), jnp.float32),
            pltpu.VMEM((epw,), jnp.int32),
            pltpu.VMEM((16 * _TPW,), jnp.float32),
            pltpu.VMEM((_TPW,), jnp.float32),
            pltpu.VMEM((_GCH,), jnp.int32),
            pltpu.VMEM((_GCH, 16), jnp.float32),
            pltpu.SemaphoreType.DMA,
        ],
    )
    return f(xp, ei, table)


def _sc_scatter(ei3, g, D, chunk, nbuf, zrows):
    nz = _TPW // zrows
    nchk = _EPW // chunk

    def body(ei_hbm, g_hbm, out, acc, sidx_all, didx_all, rows, zbuf, *sems):
        c = lax.axis_index("c")
        s = lax.axis_index("s")
        wid = s * 2 + c

        pltpu.sync_copy(ei_hbm.at[0, pl.ds(wid * nchk, nchk)], sidx_all)
        pltpu.sync_copy(ei_hbm.at[1, pl.ds(wid * nchk, nchk)], didx_all)

        def fill_zero(i, carry):
            for j in range(D // 16):
                zbuf[i, pl.ds(j * 16, 16)] = jnp.zeros((16,), jnp.float32)
            return carry

        lax.fori_loop(0, zrows, fill_zero, 0)
        for k in range(nz):
            pltpu.sync_copy(zbuf, acc.at[pl.ds(s * _TPW + k * zrows, zrows)])
        plsc.subcore_barrier()

        def gather_start(t, b):
            pltpu.make_async_copy(
                g_hbm.at[sidx_all.at[t]], rows.at[b], sems[b]).start()

        def gather_wait(t, b):
            pltpu.make_async_copy(
                g_hbm.at[sidx_all.at[t]], rows.at[b], sems[b]).wait()

        def scatter(t, b):
            pltpu.sync_copy(rows.at[b], acc.at[didx_all.at[t]], add=True)

        _ring(nchk, nbuf, gather_start, gather_wait, scatter)

        plsc.subcore_barrier()
        pltpu.sync_copy(acc.at[pl.ds(s * _TPW, _TPW)],
                        out.at[pl.ds(c * _NPAD + s * _TPW, _TPW)])

    f = pl.kernel(
        body,
        out_type=jax.ShapeDtypeStruct((2 * _NPAD, D), jnp.float32),
        mesh=_mesh(),
        compiler_params=pltpu.CompilerParams(use_tc_tiling_on_sc=False),
        scratch_types=[
            pltpu.VMEM_SHARED((_NPAD, D), jnp.float32),
            pltpu.VMEM((nchk, chunk), jnp.int32),
            pltpu.VMEM((nchk, chunk), jnp.int32),
            pltpu.VMEM((nbuf, chunk, D), jnp.float32),
            pltpu.VMEM((zrows, D), jnp.float32),
        ] + [pltpu.SemaphoreType.DMA] * nbuf,
    )
    return f(ei3, g)


_BLK = 1280
_NB = _NPAD // _BLK


def _tc_b(agg1, W1p, b1):
    def body(a, w, b, h1_ref):
        h1_ref[...] = jnp.maximum(
            jnp.dot(a[...], w[...], preferred_element_type=jnp.float32)
            + b[...], 0.0)

    return pl.pallas_call(
        body,
        grid=(_NB,),
        in_specs=[pl.BlockSpec((_BLK, 16), lambda i: (i, 0)),
                  pl.BlockSpec((16, _H), lambda i: (0, 0)),
                  pl.BlockSpec((1, _H), lambda i: (0, 0))],
        out_specs=pl.BlockSpec((_BLK, _H), lambda i: (i, 0)),
        out_shape=jax.ShapeDtypeStruct((_NPAD, _H), jnp.float32),
    )(agg1, W1p, b1)


_BLKC = 2000


def _tc_c(agg2, W2, b2, W3r, b3):
    def body(a, w2, b2r, w3, b3r, out_ref):
        h2 = jnp.maximum(
            jnp.dot(a[...], w2[...], preferred_element_type=jnp.float32)
            + b2r[...], 0.0)
        z = jnp.sum(h2 * w3[...], axis=1, keepdims=True) + b3r[...]
        out_ref[...] = jax.nn.sigmoid(z)

    return pl.pallas_call(
        body,
        grid=(_N // _BLKC,),
        in_specs=[pl.BlockSpec((_BLKC, _H), lambda i: (i, 0)),
                  pl.BlockSpec((_H, _H), lambda i: (0, 0)),
                  pl.BlockSpec((1, _H), lambda i: (0, 0)),
                  pl.BlockSpec((1, _H), lambda i: (0, 0)),
                  pl.BlockSpec((1, 1), lambda i: (0, 0))],
        out_specs=pl.BlockSpec((_BLKC, 1), lambda i: (i, 0)),
        out_shape=jax.ShapeDtypeStruct((_N, 1), jnp.float32),
    )(agg2, W2, b2, W3r, b3)


def kernel(x, edge_index, batch, item_embedding, W1, b1, W2, b2, W3, b3):
    xp = jnp.pad(x[:, 0], (0, _NPAD - _N))
    ei80 = edge_index.reshape(2, _E // _DCH, _DCH)
    ei40 = edge_index.reshape(2, _NW * (_EPW // 40), 40)
    table = jnp.pad(item_embedding, ((0, 0), (0, 16 - _EMBED)))
    W1p = jnp.pad(W1, ((0, 16 - _EMBED), (0, 0)))

    dinv, emb = _sc_deg_emb(xp, edge_index, table)
    dinv1 = dinv[:, None]
    g1 = dinv1 * emb
    scat1f = _sc_scatter(ei80, g1, 16, 80, 6, _TPW)
    agg1 = dinv1 * (scat1f[:_NPAD] + scat1f[_NPAD:] + g1)
    h1 = _tc_b(agg1, W1p, b1.reshape(1, _H))
    g2 = dinv1 * h1
    scat2f = _sc_scatter(ei40, g2, _H, 40, 5, 16)
    agg2 = dinv1 * (scat2f[:_NPAD] + scat2f[_NPAD:] + g2)
    return _tc_c(agg2, W2, b2.reshape(1, _H), W3.reshape(1, _H),
                 b3.reshape(1, 1))[:, 0]


# speedup vs baseline: 1.0484x; 1.0484x over previous
import jax
import jax.numpy as jnp
from jax import lax
from jax.experimental import pallas as pl
from jax.experimental.pallas import tpu as pltpu
from jax.experimental.pallas import tpu_sc as plsc

_N = 10000
_VOCAB = 100
_EMBED = 10
_H = 128
_E = 320000

_NW = 32
_NPAD = 10240
_NPW = _NPAD // _NW
_GCH = 80
_EPW = _E // _NW
_TPW = _NPAD // 16


def _mesh():
    return plsc.VectorSubcoreMesh(core_axis_name="c", subcore_axis_name="s")


def _ring(nchk, nbuf, gather_start, gather_wait, scatter):
    for b in range(nbuf):
        gather_start(b, b)
    fg = (nchk - nbuf) // nbuf

    def step(grp, carry):
        for b in range(nbuf):
            t = nbuf * grp + b
            gather_wait(t, b)
            scatter(t, b)
            gather_start(t + nbuf, b)
        return carry

    lax.fori_loop(0, fg, step, 0)
    for t in range(fg * nbuf, nchk):
        b = t % nbuf
        gather_wait(t, b)
        scatter(t, b)
        if t + nbuf < nchk:
            gather_start(t + nbuf, b)


_DCH = 80
_DNCHK = 2 * _EPW // _DCH
_NGRP = _NPAD // 16 // 16


def _sc_deg_emb(xp, ei, table):
    epw = 2 * _EPW

    def body(x_hbm, ei_hbm, table_hbm, dinv_out, emb_out,
             partials, degloc, didx_all, pbuf, dinvbuf, xidx, grows, sem):
        c = lax.axis_index("c")
        s = lax.axis_index("s")
        wid = s * 2 + c
        ones16 = jnp.ones((16,), jnp.float32)

        pltpu.sync_copy(ei_hbm.at[1, pl.ds(s * epw, epw)], didx_all)

        def zero_deg(i, carry):
            degloc[pl.ds(i * 16, 16)] = jnp.zeros((16,), jnp.float32)
            return carry

        lax.fori_loop(0, _NPAD // 16, zero_deg, 0)

        def deg_step(t, carry):
            for j in range(4):
                idxv = didx_all[pl.ds(t * 64 + j * 16, 16)]
                plsc.addupdate_scatter(degloc, [idxv], ones16)
            return carry

        lax.fori_loop(0, epw // 64, deg_step, 0)
        pltpu.sync_copy(degloc, partials.at[s])

        for j in range(_NPW // _GCH):
            b = wid * _NPW + j * _GCH
            pltpu.sync_copy(x_hbm.at[pl.ds(b, _GCH)], xidx)
            pltpu.async_copy(table_hbm.at[xidx], grows, sem).wait()
            pltpu.sync_copy(grows, emb_out.at[pl.ds(b, _GCH)])

        plsc.subcore_barrier()

        for p in range(16):
            pltpu.sync_copy(partials.at[p, pl.ds(s * _TPW, _TPW)],
                            pbuf.at[pl.ds(p * _TPW, _TPW)])

        def dinv_step(grp, carry):
            d = pbuf[pl.ds(grp * 16, 16)]
            for p in range(1, 16):
                d = d + pbuf[pl.ds(p * _TPW + grp * 16, 16)]
            d = d + 1.0
            yi = 1597463007 - jnp.right_shift(plsc.bitcast(d, jnp.int32), 1)
            y = plsc.bitcast(yi, jnp.float32)
            h = d * 0.5
            for _ in range(3):
                y = y * (1.5 - h * y * y)
            dinvbuf[pl.ds(grp * 16, 16)] = y
            return carry

        lax.fori_loop(0, _TPW // 16, dinv_step, 0)

        @pl.when(c == 0)
        def _():
            pltpu.sync_copy(dinvbuf, dinv_out.at[pl.ds(s * _TPW, _TPW)])

    f = pl.kernel(
        body,
        out_type=[jax.ShapeDtypeStruct((_NPAD,), jnp.float32),
                  jax.ShapeDtypeStruct((_NPAD, 16), jnp.float32)],
        mesh=_mesh(),
        compiler_params=pltpu.CompilerParams(
            use_tc_tiling_on_sc=False, needs_layout_passes=False),
        scratch_types=[
            pltpu.VMEM_SHARED((16, _NPAD), jnp.float32),
            pltpu.VMEM((_NPAD,), jnp.float32),
            pltpu.VMEM((epw,), jnp.int32),
            pltpu.VMEM((16 * _TPW,), jnp.float32),
            pltpu.VMEM((_TPW,), jnp.float32),
            pltpu.VMEM((_GCH,), jnp.int32),
            pltpu.VMEM((_GCH, 16), jnp.float32),
            pltpu.SemaphoreType.DMA,
        ],
    )
    return f(xp, ei, table)


def _sc_scatter(ei3, g, D, chunk, nbuf, zrows):
    nz = _TPW // zrows
    nchk = _EPW // chunk

    def body(ei_hbm, g_hbm, out, acc, sidx_all, didx_all, rows, zbuf, *sems):
        c = lax.axis_index("c")
        s = lax.axis_index("s")
        wid = s * 2 + c

        pltpu.sync_copy(ei_hbm.at[0, pl.ds(wid * nchk, nchk)], sidx_all)
        pltpu.sync_copy(ei_hbm.at[1, pl.ds(wid * nchk, nchk)], didx_all)

        def fill_zero(i, carry):
            for j in range(D // 16):
                zbuf[i, pl.ds(j * 16, 16)] = jnp.zeros((16,), jnp.float32)
            return carry

        lax.fori_loop(0, zrows, fill_zero, 0)
        for k in range(nz):
            pltpu.sync_copy(zbuf, acc.at[pl.ds(s * _TPW + k * zrows, zrows)])
        plsc.subcore_barrier()

        def gather_start(t, b):
            pltpu.make_async_copy(
                g_hbm.at[sidx_all.at[t]], rows.at[b], sems[b]).start()

        def gather_wait(t, b):
            pltpu.make_async_copy(
                g_hbm.at[sidx_all.at[t]], rows.at[b], sems[b]).wait()

        def scatter(t, b):
            pltpu.sync_copy(rows.at[b], acc.at[didx_all.at[t]], add=True)

        _ring(nchk, nbuf, gather_start, gather_wait, scatter)

        plsc.subcore_barrier()
        pltpu.sync_copy(acc.at[pl.ds(s * _TPW, _TPW)],
                        out.at[pl.ds(c * _NPAD + s * _TPW, _TPW)])

    f = pl.kernel(
        body,
        out_type=jax.ShapeDtypeStruct((2 * _NPAD, D), jnp.float32),
        mesh=_mesh(),
        compiler_params=pltpu.CompilerParams(use_tc_tiling_on_sc=False),
        scratch_types=[
            pltpu.VMEM_SHARED((_NPAD, D), jnp.float32),
            pltpu.VMEM((nchk, chunk), jnp.int32),
            pltpu.VMEM((nchk, chunk), jnp.int32),
            pltpu.VMEM((nbuf, chunk, D), jnp.float32),
            pltpu.VMEM((zrows, D), jnp.float32),
        ] + [pltpu.SemaphoreType.DMA] * nbuf,
    )
    return f(ei3, g)


_BLK = 1280
_NB = _NPAD // _BLK


def _tc_b(agg1, W1p, b1):
    def body(a, w, b, h1_ref):
        h1_ref[...] = jnp.maximum(
            jnp.dot(a[...], w[...], preferred_element_type=jnp.float32)
            + b[...], 0.0)

    return pl.pallas_call(
        body,
        grid=(_NB,),
        in_specs=[pl.BlockSpec((_BLK, 16), lambda i: (i, 0)),
                  pl.BlockSpec((16, _H), lambda i: (0, 0)),
                  pl.BlockSpec((1, _H), lambda i: (0, 0))],
        out_specs=pl.BlockSpec((_BLK, _H), lambda i: (i, 0)),
        out_shape=jax.ShapeDtypeStruct((_NPAD, _H), jnp.float32),
    )(agg1, W1p, b1)


_BLKC = 2000


def _tc_c(agg2, W2, b2, W3r, b3):
    def body(a, w2, b2r, w3, b3r, out_ref):
        h2 = jnp.maximum(
            jnp.dot(a[...], w2[...], preferred_element_type=jnp.float32)
            + b2r[...], 0.0)
        z = jnp.sum(h2 * w3[...], axis=1, keepdims=True) + b3r[...]
        out_ref[...] = jax.nn.sigmoid(z)

    return pl.pallas_call(
        body,
        grid=(_N // _BLKC,),
        in_specs=[pl.BlockSpec((_BLKC, _H), lambda i: (i, 0)),
                  pl.BlockSpec((_H, _H), lambda i: (0, 0)),
                  pl.BlockSpec((1, _H), lambda i: (0, 0)),
                  pl.BlockSpec((1, _H), lambda i: (0, 0)),
                  pl.BlockSpec((1, 1), lambda i: (0, 0))],
        out_specs=pl.BlockSpec((_BLKC, 1), lambda i: (i, 0)),
        out_shape=jax.ShapeDtypeStruct((_N, 1), jnp.float32),
    )(agg2, W2, b2, W3r, b3)


def kernel(x, edge_index, batch, item_embedding, W1, b1, W2, b2, W3, b3):
    xp = jnp.pad(x[:, 0], (0, _NPAD - _N))
    ei80 = edge_index.reshape(2, _E // _DCH, _DCH)
    ei40 = edge_index.reshape(2, _NW * (_EPW // 40), 40)
    table = jnp.pad(item_embedding, ((0, 0), (0, 16 - _EMBED)))
    W1p = jnp.pad(W1, ((0, 16 - _EMBED), (0, 0)))

    dinv, emb = _sc_deg_emb(xp, edge_index, table)
    dinv1 = dinv[:, None]
    g1 = dinv1 * emb
    scat1f = _sc_scatter(ei80, g1, 16, 80, 6, _TPW)
    agg1 = dinv1 * (scat1f[:_NPAD] + scat1f[_NPAD:] + g1)
    h1 = _tc_b(agg1, W1p, b1.reshape(1, _H))
    g2 = dinv1 * h1
    scat2f = _sc_scatter(ei40, g2, _H, 40, 5, 16)
    agg2 = dinv1 * (scat2f[:_NPAD] + scat2f[_NPAD:] + g2)
    return _tc_c(agg2, W2, b2.reshape(1, _H), W3.reshape(1, _H),
                 b3.reshape(1, 1))[:, 0]


# speedup vs baseline: 1.1048x; 1.0538x over previous
import jax
import jax.numpy as jnp
from jax import lax
from jax.experimental import pallas as pl
from jax.experimental.pallas import tpu as pltpu
from jax.experimental.pallas import tpu_sc as plsc

_N = 10000
_VOCAB = 100
_EMBED = 10
_H = 128
_E = 320000

_NW = 32
_NPAD = 10240
_NPW = _NPAD // _NW
_GCH = 80
_EPW = _E // _NW
_TPW = _NPAD // 16


def _mesh():
    return plsc.VectorSubcoreMesh(core_axis_name="c", subcore_axis_name="s")


def _ring(nchk, nbuf, gather_start, gather_wait, scatter):
    for b in range(nbuf):
        gather_start(b, b)
    fg = (nchk - nbuf) // nbuf

    def step(grp, carry):
        for b in range(nbuf):
            t = nbuf * grp + b
            gather_wait(t, b)
            scatter(t, b)
            gather_start(t + nbuf, b)
        return carry

    lax.fori_loop(0, fg, step, 0)
    for t in range(fg * nbuf, nchk):
        b = t % nbuf
        gather_wait(t, b)
        scatter(t, b)
        if t + nbuf < nchk:
            gather_start(t + nbuf, b)


_DCH = 80
_DNCHK = 2 * _EPW // _DCH
_NGRP = _NPAD // 16 // 16


def _sc_deg_emb(xp, ei, table):
    epw = 2 * _EPW

    def body(x_hbm, ei_hbm, table_hbm, dinv_out, g1_out,
             partials, degloc, didx_all, pbuf, dinvbuf, xloc, tabflat,
             g1buf):
        c = lax.axis_index("c")
        s = lax.axis_index("s")
        wid = s * 2 + c
        iota = lax.iota(jnp.int32, 16)
        ones16 = jnp.ones((16,), jnp.float32)

        pltpu.sync_copy(ei_hbm.at[1, pl.ds(s * epw, epw)], didx_all)

        def zero_deg(i, carry):
            degloc[pl.ds(i * 16, 16)] = jnp.zeros((16,), jnp.float32)
            return carry

        lax.fori_loop(0, _NPAD // 16, zero_deg, 0)

        def deg_step(t, carry):
            for j in range(4):
                idxv = didx_all[pl.ds(t * 64 + j * 16, 16)]
                plsc.addupdate_scatter(degloc, [idxv], ones16)
            return carry

        lax.fori_loop(0, epw // 64, deg_step, 0)
        pltpu.sync_copy(degloc, partials.at[s])

        pltpu.sync_copy(x_hbm.at[pl.ds(wid * _NPW, _NPW)], xloc)
        pltpu.sync_copy(table_hbm, tabflat)

        plsc.subcore_barrier()

        for p in range(16):
            pltpu.sync_copy(partials.at[p, pl.ds(s * _TPW, _TPW)],
                            pbuf.at[pl.ds(p * _TPW, _TPW)])

        def dinv_step(grp, carry):
            d = pbuf[pl.ds(grp * 16, 16)]
            for p in range(1, 16):
                d = d + pbuf[pl.ds(p * _TPW + grp * 16, 16)]
            d = d + 1.0
            yi = 1597463007 - jnp.right_shift(plsc.bitcast(d, jnp.int32), 1)
            y = plsc.bitcast(yi, jnp.float32)
            h = d * 0.5
            for _ in range(3):
                y = y * (1.5 - h * y * y)
            dinvbuf[pl.ds(grp * 16, 16)] = y
            return carry

        lax.fori_loop(0, _TPW // 16, dinv_step, 0)

        @pl.when(c == 0)
        def _():
            pltpu.sync_copy(dinvbuf, dinv_out.at[pl.ds(s * _TPW, _TPW)])

        def scale_step(grp, carry):
            xv = xloc[pl.ds(grp * 16, 16)]
            dv = dinvbuf[pl.ds(c * _NPW + grp * 16, 16)]
            tbase = xv * 16
            obase = (grp * 16 + iota) * 16
            for j in range(16):
                col = plsc.load_gather(tabflat, [tbase + j])
                plsc.store_scatter(g1buf, [obase + j], col * dv)
            return carry

        lax.fori_loop(0, _NPW // 16, scale_step, 0)
        pltpu.sync_copy(g1buf, g1_out.at[pl.ds(wid * _NPW * 16, _NPW * 16)])

    f = pl.kernel(
        body,
        out_type=[jax.ShapeDtypeStruct((_NPAD,), jnp.float32),
                  jax.ShapeDtypeStruct((_NPAD * 16,), jnp.float32)],
        mesh=_mesh(),
        compiler_params=pltpu.CompilerParams(
            use_tc_tiling_on_sc=False, needs_layout_passes=False),
        scratch_types=[
            pltpu.VMEM_SHARED((16, _NPAD), jnp.float32),
            pltpu.VMEM((_NPAD,), jnp.float32),
            pltpu.VMEM((epw,), jnp.int32),
            pltpu.VMEM((16 * _TPW,), jnp.float32),
            pltpu.VMEM((_TPW,), jnp.float32),
            pltpu.VMEM((_NPW,), jnp.int32),
            pltpu.VMEM((16 * _VOCAB,), jnp.float32),
            pltpu.VMEM((_NPW * 16,), jnp.float32),
        ],
    )
    return f(xp, ei, table)


def _sc_scatter(ei3, g, D, chunk, nbuf, zrows):
    nz = _TPW // zrows
    nchk = _EPW // chunk

    def body(ei_hbm, g_hbm, out, acc, sidx_all, didx_all, rows, zbuf, *sems):
        c = lax.axis_index("c")
        s = lax.axis_index("s")
        wid = s * 2 + c

        pltpu.sync_copy(ei_hbm.at[0, pl.ds(wid * nchk, nchk)], sidx_all)
        pltpu.sync_copy(ei_hbm.at[1, pl.ds(wid * nchk, nchk)], didx_all)

        def fill_zero(i, carry):
            for j in range(D // 16):
                zbuf[i, pl.ds(j * 16, 16)] = jnp.zeros((16,), jnp.float32)
            return carry

        lax.fori_loop(0, zrows, fill_zero, 0)
        for k in range(nz):
            pltpu.sync_copy(zbuf, acc.at[pl.ds(s * _TPW + k * zrows, zrows)])
        plsc.subcore_barrier()

        def gather_start(t, b):
            pltpu.make_async_copy(
                g_hbm.at[sidx_all.at[t]], rows.at[b], sems[b]).start()

        def gather_wait(t, b):
            pltpu.make_async_copy(
                g_hbm.at[sidx_all.at[t]], rows.at[b], sems[b]).wait()

        def scatter(t, b):
            pltpu.sync_copy(rows.at[b], acc.at[didx_all.at[t]], add=True)

        _ring(nchk, nbuf, gather_start, gather_wait, scatter)

        plsc.subcore_barrier()
        pltpu.sync_copy(acc.at[pl.ds(s * _TPW, _TPW)],
                        out.at[pl.ds(c * _NPAD + s * _TPW, _TPW)])

    f = pl.kernel(
        body,
        out_type=jax.ShapeDtypeStruct((2 * _NPAD, D), jnp.float32),
        mesh=_mesh(),
        compiler_params=pltpu.CompilerParams(use_tc_tiling_on_sc=False),
        scratch_types=[
            pltpu.VMEM_SHARED((_NPAD, D), jnp.float32),
            pltpu.VMEM((nchk, chunk), jnp.int32),
            pltpu.VMEM((nchk, chunk), jnp.int32),
            pltpu.VMEM((nbuf, chunk, D), jnp.float32),
            pltpu.VMEM((zrows, D), jnp.float32),
        ] + [pltpu.SemaphoreType.DMA] * nbuf,
    )
    return f(ei3, g)


_BLK = 1280
_NB = _NPAD // _BLK


def _tc_b(agg1, W1p, b1):
    def body(a, w, b, h1_ref):
        h1_ref[...] = jnp.maximum(
            jnp.dot(a[...], w[...], preferred_element_type=jnp.float32)
            + b[...], 0.0)

    return pl.pallas_call(
        body,
        grid=(_NB,),
        in_specs=[pl.BlockSpec((_BLK, 16), lambda i: (i, 0)),
                  pl.BlockSpec((16, _H), lambda i: (0, 0)),
                  pl.BlockSpec((1, _H), lambda i: (0, 0))],
        out_specs=pl.BlockSpec((_BLK, _H), lambda i: (i, 0)),
        out_shape=jax.ShapeDtypeStruct((_NPAD, _H), jnp.float32),
    )(agg1, W1p, b1)


_BLKC = 2000


def _tc_c(agg2, W2, b2, W3r, b3):
    def body(a, w2, b2r, w3, b3r, out_ref):
        h2 = jnp.maximum(
            jnp.dot(a[...], w2[...], preferred_element_type=jnp.float32)
            + b2r[...], 0.0)
        z = jnp.sum(h2 * w3[...], axis=1, keepdims=True) + b3r[...]
        out_ref[...] = jax.nn.sigmoid(z)

    return pl.pallas_call(
        body,
        grid=(_N // _BLKC,),
        in_specs=[pl.BlockSpec((_BLKC, _H), lambda i: (i, 0)),
                  pl.BlockSpec((_H, _H), lambda i: (0, 0)),
                  pl.BlockSpec((1, _H), lambda i: (0, 0)),
                  pl.BlockSpec((1, _H), lambda i: (0, 0)),
                  pl.BlockSpec((1, 1), lambda i: (0, 0))],
        out_specs=pl.BlockSpec((_BLKC, 1), lambda i: (i, 0)),
        out_shape=jax.ShapeDtypeStruct((_N, 1), jnp.float32),
    )(agg2, W2, b2, W3r, b3)


def kernel(x, edge_index, batch, item_embedding, W1, b1, W2, b2, W3, b3):
    xp = jnp.pad(x[:, 0], (0, _NPAD - _N))
    ei80 = edge_index.reshape(2, _E // _DCH, _DCH)
    ei40 = edge_index.reshape(2, _NW * (_EPW // 40), 40)
    table = jnp.pad(item_embedding, ((0, 0), (0, 16 - _EMBED)))
    W1p = jnp.pad(W1, ((0, 16 - _EMBED), (0, 0)))

    dinv, g1f = _sc_deg_emb(xp, edge_index, table.reshape(-1))
    dinv1 = dinv[:, None]
    g1 = g1f.reshape(_NPAD, 16)
    scat1f = _sc_scatter(ei80, g1, 16, 80, 6, _TPW)
    agg1 = dinv1 * (scat1f[:_NPAD] + scat1f[_NPAD:] + g1)
    h1 = _tc_b(agg1, W1p, b1.reshape(1, _H))
    g2 = dinv1 * h1
    scat2f = _sc_scatter(ei40, g2, _H, 40, 5, 16)
    agg2 = dinv1 * (scat2f[:_NPAD] + scat2f[_NPAD:] + g2)
    return _tc_c(agg2, W2, b2.reshape(1, _H), W3.reshape(1, _H),
                 b3.reshape(1, 1))[:, 0]


# speedup vs baseline: 1.1224x; 1.0159x over previous
import jax
import jax.numpy as jnp
from jax import lax
from jax.experimental import pallas as pl
from jax.experimental.pallas import tpu as pltpu
from jax.experimental.pallas import tpu_sc as plsc

_N = 10000
_VOCAB = 100
_EMBED = 10
_H = 128
_E = 320000

_NW = 32
_NPAD = 10240
_NPW = _NPAD // _NW
_GCH = 80
_EPW = _E // _NW
_TPW = _NPAD // 16


def _mesh():
    return plsc.VectorSubcoreMesh(core_axis_name="c", subcore_axis_name="s")


def _ring(nchk, nbuf, gather_start, gather_wait, scatter):
    for b in range(nbuf):
        gather_start(b, b)
    fg = (nchk - nbuf) // nbuf

    def step(grp, carry):
        for b in range(nbuf):
            t = nbuf * grp + b
            gather_wait(t, b)
            scatter(t, b)
            gather_start(t + nbuf, b)
        return carry

    lax.fori_loop(0, fg, step, 0)
    for t in range(fg * nbuf, nchk):
        b = t % nbuf
        gather_wait(t, b)
        scatter(t, b)
        if t + nbuf < nchk:
            gather_start(t + nbuf, b)


_DCH = 80
_DNCHK = 2 * _EPW // _DCH
_NGRP = _NPAD // 16 // 16


def _sc_deg_emb(xp, ei, table):
    epw = 2 * _EPW

    def body(x_hbm, ei_hbm, table_hbm, dinv_out, g1_out,
             partials, degloc, didx_all, pbuf, dinvbuf, xloc, tabflat,
             g1buf):
        c = lax.axis_index("c")
        s = lax.axis_index("s")
        wid = s * 2 + c
        iota = lax.iota(jnp.int32, 16)
        ones16 = jnp.ones((16,), jnp.float32)

        pltpu.sync_copy(ei_hbm.at[1, pl.ds(s * epw, epw)], didx_all)

        def zero_deg(i, carry):
            degloc[pl.ds(i * 16, 16)] = jnp.zeros((16,), jnp.float32)
            return carry

        lax.fori_loop(0, _NPAD // 16, zero_deg, 0)

        def deg_step(t, carry):
            for j in range(4):
                idxv = didx_all[pl.ds(t * 64 + j * 16, 16)]
                plsc.addupdate_scatter(degloc, [idxv], ones16)
            return carry

        lax.fori_loop(0, epw // 64, deg_step, 0)
        pltpu.sync_copy(degloc, partials.at[s])

        pltpu.sync_copy(x_hbm.at[pl.ds(wid * _NPW, _NPW)], xloc)
        pltpu.sync_copy(table_hbm, tabflat)

        plsc.subcore_barrier()

        for p in range(16):
            pltpu.sync_copy(partials.at[p, pl.ds(s * _TPW, _TPW)],
                            pbuf.at[pl.ds(p * _TPW, _TPW)])

        def dinv_step(grp, carry):
            d = pbuf[pl.ds(grp * 16, 16)]
            for p in range(1, 16):
                d = d + pbuf[pl.ds(p * _TPW + grp * 16, 16)]
            d = d + 1.0
            yi = 1597463007 - jnp.right_shift(plsc.bitcast(d, jnp.int32), 1)
            y = plsc.bitcast(yi, jnp.float32)
            h = d * 0.5
            for _ in range(3):
                y = y * (1.5 - h * y * y)
            dinvbuf[pl.ds(grp * 16, 16)] = y
            return carry

        lax.fori_loop(0, _TPW // 16, dinv_step, 0)

        @pl.when(c == 0)
        def _():
            pltpu.sync_copy(dinvbuf, dinv_out.at[pl.ds(s * _TPW, _TPW)])

        def scale_step(grp, carry):
            xv = xloc[pl.ds(grp * 16, 16)]
            dv = dinvbuf[pl.ds(c * _NPW + grp * 16, 16)]
            tbase = xv * 16
            obase = (grp * 16 + iota) * 16
            for j in range(16):
                col = plsc.load_gather(tabflat, [tbase + j])
                plsc.store_scatter(g1buf, [obase + j], col * dv)
            return carry

        lax.fori_loop(0, _NPW // 16, scale_step, 0)
        pltpu.sync_copy(g1buf, g1_out.at[pl.ds(wid * _NPW * 16, _NPW * 16)])

    f = pl.kernel(
        body,
        out_type=[jax.ShapeDtypeStruct((_NPAD,), jnp.float32),
                  jax.ShapeDtypeStruct((_NPAD * 16,), jnp.float32)],
        mesh=_mesh(),
        compiler_params=pltpu.CompilerParams(
            use_tc_tiling_on_sc=False, needs_layout_passes=False),
        scratch_types=[
            pltpu.VMEM_SHARED((16, _NPAD), jnp.float32),
            pltpu.VMEM((_NPAD,), jnp.float32),
            pltpu.VMEM((epw,), jnp.int32),
            pltpu.VMEM((16 * _TPW,), jnp.float32),
            pltpu.VMEM((_TPW,), jnp.float32),
            pltpu.VMEM((_NPW,), jnp.int32),
            pltpu.VMEM((16 * _VOCAB,), jnp.float32),
            pltpu.VMEM((_NPW * 16,), jnp.float32),
        ],
    )
    return f(xp, ei, table)


def _sc_scatter(ei3, g, D, chunk, nbuf, zrows):
    nz = _TPW // zrows
    nchk = _EPW // chunk

    def body(ei_hbm, g_hbm, out, acc, sidx_all, didx_all, rows, zbuf, *sems):
        c = lax.axis_index("c")
        s = lax.axis_index("s")
        wid = s * 2 + c

        pltpu.sync_copy(ei_hbm.at[0, pl.ds(wid * nchk, nchk)], sidx_all)
        pltpu.sync_copy(ei_hbm.at[1, pl.ds(wid * nchk, nchk)], didx_all)

        def fill_zero(i, carry):
            for j in range(D // 16):
                zbuf[i, pl.ds(j * 16, 16)] = jnp.zeros((16,), jnp.float32)
            return carry

        lax.fori_loop(0, zrows, fill_zero, 0)
        for k in range(nz):
            pltpu.sync_copy(zbuf, acc.at[pl.ds(s * _TPW + k * zrows, zrows)])
        plsc.subcore_barrier()

        def gather_start(t, b):
            pltpu.make_async_copy(
                g_hbm.at[sidx_all.at[t]], rows.at[b], sems[b]).start()

        def gather_wait(t, b):
            pltpu.make_async_copy(
                g_hbm.at[sidx_all.at[t]], rows.at[b], sems[b]).wait()

        def scatter(t, b):
            pltpu.sync_copy(rows.at[b], acc.at[didx_all.at[t]], add=True)

        _ring(nchk, nbuf, gather_start, gather_wait, scatter)

        plsc.subcore_barrier()
        pltpu.sync_copy(acc.at[pl.ds(s * _TPW, _TPW)],
                        out.at[pl.ds(c * _NPAD + s * _TPW, _TPW)])

    f = pl.kernel(
        body,
        out_type=jax.ShapeDtypeStruct((2 * _NPAD, D), jnp.float32),
        mesh=_mesh(),
        compiler_params=pltpu.CompilerParams(use_tc_tiling_on_sc=False),
        scratch_types=[
            pltpu.VMEM_SHARED((_NPAD, D), jnp.float32),
            pltpu.VMEM((nchk, chunk), jnp.int32),
            pltpu.VMEM((nchk, chunk), jnp.int32),
            pltpu.VMEM((nbuf, chunk, D), jnp.float32),
            pltpu.VMEM((zrows, D), jnp.float32),
        ] + [pltpu.SemaphoreType.DMA] * nbuf,
    )
    return f(ei3, g)


_BLK = 1280
_NB = _NPAD // _BLK


def _tc_b(agg1, W1p, b1):
    def body(a, w, b, h1_ref):
        h1_ref[...] = jnp.maximum(
            jnp.dot(a[...], w[...], preferred_element_type=jnp.float32)
            + b[...], 0.0)

    return pl.pallas_call(
        body,
        grid=(_NB,),
        in_specs=[pl.BlockSpec((_BLK, 16), lambda i: (i, 0)),
                  pl.BlockSpec((16, _H), lambda i: (0, 0)),
                  pl.BlockSpec((1, _H), lambda i: (0, 0))],
        out_specs=pl.BlockSpec((_BLK, _H), lambda i: (i, 0)),
        out_shape=jax.ShapeDtypeStruct((_NPAD, _H), jnp.float32),
    )(agg1, W1p, b1)


_BLKC = 2000


def _tc_c(agg2, W2, b2, W3r, b3):
    def body(a, w2, b2r, w3, b3r, out_ref):
        h2 = jnp.maximum(
            jnp.dot(a[...], w2[...], preferred_element_type=jnp.float32)
            + b2r[...], 0.0)
        z = jnp.sum(h2 * w3[...], axis=1, keepdims=True) + b3r[...]
        out_ref[...] = jax.nn.sigmoid(z)

    return pl.pallas_call(
        body,
        grid=(_N // _BLKC,),
        in_specs=[pl.BlockSpec((_BLKC, _H), lambda i: (i, 0)),
                  pl.BlockSpec((_H, _H), lambda i: (0, 0)),
                  pl.BlockSpec((1, _H), lambda i: (0, 0)),
                  pl.BlockSpec((1, _H), lambda i: (0, 0)),
                  pl.BlockSpec((1, 1), lambda i: (0, 0))],
        out_specs=pl.BlockSpec((_BLKC, 1), lambda i: (i, 0)),
        out_shape=jax.ShapeDtypeStruct((_N, 1), jnp.float32),
    )(agg2, W2, b2, W3r, b3)


def kernel(x, edge_index, batch, item_embedding, W1, b1, W2, b2, W3, b3):
    xp = jnp.pad(x[:, 0], (0, _NPAD - _N))
    ei80 = edge_index.reshape(2, _E // _DCH, _DCH)
    ei40 = edge_index.reshape(2, _NW * (_EPW // 40), 40)
    table = jnp.pad(item_embedding, ((0, 0), (0, 16 - _EMBED)))
    W1p = jnp.pad(W1, ((0, 16 - _EMBED), (0, 0)))

    dinv, g1f = _sc_deg_emb(xp, edge_index, table.reshape(-1))
    dinv1 = dinv[:, None]
    g1 = g1f.reshape(_NPAD, 16)
    scat1f = _sc_scatter(ei80, g1, 16, 80, 8, _TPW)
    agg1 = dinv1 * (scat1f[:_NPAD] + scat1f[_NPAD:] + g1)
    h1 = _tc_b(agg1, W1p, b1.reshape(1, _H))
    g2 = dinv1 * h1
    scat2f = _sc_scatter(ei40, g2, _H, 40, 5, 16)
    agg2 = dinv1 * (scat2f[:_NPAD] + scat2f[_NPAD:] + g2)
    return _tc_c(agg2, W2, b2.reshape(1, _H), W3.reshape(1, _H),
                 b3.reshape(1, 1))[:, 0]


# speedup vs baseline: 1.1284x; 1.0054x over previous
import jax
import jax.numpy as jnp
from jax import lax
from jax.experimental import pallas as pl
from jax.experimental.pallas import tpu as pltpu
from jax.experimental.pallas import tpu_sc as plsc

_N = 10000
_VOCAB = 100
_EMBED = 10
_H = 128
_E = 320000

_NW = 32
_NPAD = 10240
_NPW = _NPAD // _NW
_GCH = 80
_EPW = _E // _NW
_TPW = _NPAD // 16


def _mesh():
    return plsc.VectorSubcoreMesh(core_axis_name="c", subcore_axis_name="s")


def _ring(nchk, nbuf, gather_start, gather_wait, scatter):
    for b in range(nbuf):
        gather_start(b, b)
    fg = (nchk - nbuf) // nbuf

    def step(grp, carry):
        for b in range(nbuf):
            t = nbuf * grp + b
            gather_wait(t, b)
            scatter(t, b)
            gather_start(t + nbuf, b)
        return carry

    lax.fori_loop(0, fg, step, 0)
    for t in range(fg * nbuf, nchk):
        b = t % nbuf
        gather_wait(t, b)
        scatter(t, b)
        if t + nbuf < nchk:
            gather_start(t + nbuf, b)


_DCH = 80
_DNCHK = 2 * _EPW // _DCH
_NGRP = _NPAD // 16 // 16


def _sc_deg_emb(xp, ei, table):
    epw = 2 * _EPW

    def body(x_hbm, ei_hbm, table_hbm, dinv_out, g1_out,
             partials, degloc, didx_all, pbuf, dinvbuf, xloc, tabflat,
             g1buf):
        c = lax.axis_index("c")
        s = lax.axis_index("s")
        wid = s * 2 + c
        iota = lax.iota(jnp.int32, 16)
        ones16 = jnp.ones((16,), jnp.float32)

        pltpu.sync_copy(ei_hbm.at[1, pl.ds(s * epw, epw)], didx_all)

        def zero_deg(i, carry):
            degloc[pl.ds(i * 16, 16)] = jnp.zeros((16,), jnp.float32)
            return carry

        lax.fori_loop(0, _NPAD // 16, zero_deg, 0)

        def deg_step(t, carry):
            for j in range(4):
                idxv = didx_all[pl.ds(t * 64 + j * 16, 16)]
                plsc.addupdate_scatter(degloc, [idxv], ones16)
            return carry

        lax.fori_loop(0, epw // 64, deg_step, 0)
        pltpu.sync_copy(degloc, partials.at[s])

        pltpu.sync_copy(x_hbm.at[pl.ds(wid * _NPW, _NPW)], xloc)
        pltpu.sync_copy(table_hbm, tabflat)

        plsc.subcore_barrier()

        for p in range(16):
            pltpu.sync_copy(partials.at[p, pl.ds(s * _TPW, _TPW)],
                            pbuf.at[pl.ds(p * _TPW, _TPW)])

        def dinv_step(grp, carry):
            d = pbuf[pl.ds(grp * 16, 16)]
            for p in range(1, 16):
                d = d + pbuf[pl.ds(p * _TPW + grp * 16, 16)]
            d = d + 1.0
            yi = 1597463007 - jnp.right_shift(plsc.bitcast(d, jnp.int32), 1)
            y = plsc.bitcast(yi, jnp.float32)
            h = d * 0.5
            for _ in range(3):
                y = y * (1.5 - h * y * y)
            dinvbuf[pl.ds(grp * 16, 16)] = y
            return carry

        lax.fori_loop(0, _TPW // 16, dinv_step, 0)

        @pl.when(c == 0)
        def _():
            pltpu.sync_copy(dinvbuf, dinv_out.at[pl.ds(s * _TPW, _TPW)])

        def scale_step(grp, carry):
            xv = xloc[pl.ds(grp * 16, 16)]
            dv = dinvbuf[pl.ds(c * _NPW + grp * 16, 16)]
            tbase = xv * 16
            obase = (grp * 16 + iota) * 16
            for j in range(16):
                col = plsc.load_gather(tabflat, [tbase + j])
                plsc.store_scatter(g1buf, [obase + j], col * dv)
            return carry

        lax.fori_loop(0, _NPW // 16, scale_step, 0)
        pltpu.sync_copy(g1buf, g1_out.at[pl.ds(wid * _NPW * 16, _NPW * 16)])

    f = pl.kernel(
        body,
        out_type=[jax.ShapeDtypeStruct((_NPAD,), jnp.float32),
                  jax.ShapeDtypeStruct((_NPAD * 16,), jnp.float32)],
        mesh=_mesh(),
        compiler_params=pltpu.CompilerParams(
            use_tc_tiling_on_sc=False, needs_layout_passes=False),
        scratch_types=[
            pltpu.VMEM_SHARED((16, _NPAD), jnp.float32),
            pltpu.VMEM((_NPAD,), jnp.float32),
            pltpu.VMEM((epw,), jnp.int32),
            pltpu.VMEM((16 * _TPW,), jnp.float32),
            pltpu.VMEM((_TPW,), jnp.float32),
            pltpu.VMEM((_NPW,), jnp.int32),
            pltpu.VMEM((16 * _VOCAB,), jnp.float32),
            pltpu.VMEM((_NPW * 16,), jnp.float32),
        ],
    )
    return f(xp, ei, table)


def _sc_scatter(ei3, g, D, chunk, nbuf, zrows):
    nz = _TPW // zrows
    nchk = _EPW // chunk

    def body(ei_hbm, g_hbm, out, acc, sidx_all, didx_all, rows, zbuf, *sems):
        c = lax.axis_index("c")
        s = lax.axis_index("s")
        wid = s * 2 + c

        pltpu.sync_copy(ei_hbm.at[0, pl.ds(wid * nchk, nchk)], sidx_all)
        pltpu.sync_copy(ei_hbm.at[1, pl.ds(wid * nchk, nchk)], didx_all)

        def fill_zero(i, carry):
            for j in range(D // 16):
                zbuf[i, pl.ds(j * 16, 16)] = jnp.zeros((16,), jnp.float32)
            return carry

        lax.fori_loop(0, zrows, fill_zero, 0)
        for k in range(nz):
            pltpu.sync_copy(zbuf, acc.at[pl.ds(s * _TPW + k * zrows, zrows)])
        plsc.subcore_barrier()

        def gather_start(t, b):
            pltpu.make_async_copy(
                g_hbm.at[sidx_all.at[t]], rows.at[b], sems[b]).start()

        def gather_wait(t, b):
            pltpu.make_async_copy(
                g_hbm.at[sidx_all.at[t]], rows.at[b], sems[b]).wait()

        def scatter(t, b):
            pltpu.sync_copy(rows.at[b], acc.at[didx_all.at[t]], add=True)

        _ring(nchk, nbuf, gather_start, gather_wait, scatter)

        plsc.subcore_barrier()
        pltpu.sync_copy(acc.at[pl.ds(s * _TPW, _TPW)],
                        out.at[pl.ds(c * _NPAD + s * _TPW, _TPW)])

    f = pl.kernel(
        body,
        out_type=jax.ShapeDtypeStruct((2 * _NPAD, D), jnp.float32),
        mesh=_mesh(),
        compiler_params=pltpu.CompilerParams(use_tc_tiling_on_sc=False),
        scratch_types=[
            pltpu.VMEM_SHARED((_NPAD, D), jnp.float32),
            pltpu.VMEM((nchk, chunk), jnp.int32),
            pltpu.VMEM((nchk, chunk), jnp.int32),
            pltpu.VMEM((nbuf, chunk, D), jnp.float32),
            pltpu.VMEM((zrows, D), jnp.float32),
        ] + [pltpu.SemaphoreType.DMA] * nbuf,
    )
    return f(ei3, g)


_BLK = 1280
_NB = _NPAD // _BLK


def _tc_b(agg1, W1p, b1):
    def body(a, w, b, h1_ref):
        h1_ref[...] = jnp.maximum(
            jnp.dot(a[...], w[...], preferred_element_type=jnp.float32)
            + b[...], 0.0)

    return pl.pallas_call(
        body,
        grid=(_NB,),
        in_specs=[pl.BlockSpec((_BLK, 16), lambda i: (i, 0)),
                  pl.BlockSpec((16, _H), lambda i: (0, 0)),
                  pl.BlockSpec((1, _H), lambda i: (0, 0))],
        out_specs=pl.BlockSpec((_BLK, _H), lambda i: (i, 0)),
        out_shape=jax.ShapeDtypeStruct((_NPAD, _H), jnp.float32),
    )(agg1, W1p, b1)


_BLKC = 2000


def _tc_c(agg2, W2, b2, W3r, b3):
    def body(a, w2, b2r, w3, b3r, out_ref):
        h2 = jnp.maximum(
            jnp.dot(a[...], w2[...], preferred_element_type=jnp.float32)
            + b2r[...], 0.0)
        z = jnp.sum(h2 * w3[...], axis=1, keepdims=True) + b3r[...]
        out_ref[...] = jax.nn.sigmoid(z)

    return pl.pallas_call(
        body,
        grid=(_N // _BLKC,),
        in_specs=[pl.BlockSpec((_BLKC, _H), lambda i: (i, 0)),
                  pl.BlockSpec((_H, _H), lambda i: (0, 0)),
                  pl.BlockSpec((1, _H), lambda i: (0, 0)),
                  pl.BlockSpec((1, _H), lambda i: (0, 0)),
                  pl.BlockSpec((1, 1), lambda i: (0, 0))],
        out_specs=pl.BlockSpec((_BLKC, 1), lambda i: (i, 0)),
        out_shape=jax.ShapeDtypeStruct((_N, 1), jnp.float32),
    )(agg2, W2, b2, W3r, b3)


def kernel(x, edge_index, batch, item_embedding, W1, b1, W2, b2, W3, b3):
    xp = jnp.pad(x[:, 0], (0, _NPAD - _N))
    ei80 = edge_index.reshape(2, _E // _DCH, _DCH)
    ei40 = edge_index.reshape(2, _NW * (_EPW // 40), 40)
    table = jnp.pad(item_embedding, ((0, 0), (0, 16 - _EMBED)))
    W1p = jnp.pad(W1, ((0, 16 - _EMBED), (0, 0)))

    dinv, g1f = _sc_deg_emb(xp, edge_index, table.reshape(-1))
    dinv1 = dinv[:, None]
    g1 = g1f.reshape(_NPAD, 16)
    scat1f = _sc_scatter(ei80, g1, 16, 80, 10, _TPW)
    agg1 = dinv1 * (scat1f[:_NPAD] + scat1f[_NPAD:] + g1)
    h1 = _tc_b(agg1, W1p, b1.reshape(1, _H))
    g2 = dinv1 * h1
    scat2f = _sc_scatter(ei40, g2, _H, 40, 5, 16)
    agg2 = dinv1 * (scat2f[:_NPAD] + scat2f[_NPAD:] + g2)
    return _tc_c(agg2, W2, b2.reshape(1, _H), W3.reshape(1, _H),
                 b3.reshape(1, 1))[:, 0]
